# Initial kernel scaffold; baseline (speedup 1.0000x reference)
import jax
import jax.numpy as jnp
from jax.experimental import pallas as pl


def kernel(seq_indices, expr_values, emb_table, expr_proj_w, expr_proj_b):
    raise NotImplementedError("write your pallas kernel here")


# speedup vs baseline: 9.7147x; 9.7147x over previous
import functools

import jax
import jax.numpy as jnp
from jax import lax
from jax.experimental import pallas as pl
from jax.experimental.pallas import tpu as pltpu
from jax.experimental.pallas import tpu_sc as plsc

EMBED_DIM = 64
NB_TOKENS = 50
ROWS = 16384
LANES = 16
NC, NS = 2, 16
NW = NC * NS
ROWS_PER_W = ROWS // NW
PAIR = 2
PAIRS_PER_BLK = 16
BLK_ROWS = PAIR * PAIRS_PER_BLK
NBLK = ROWS_PER_W // BLK_ROWS
DCH = EMBED_DIM // LANES


def _sc_kernel(idx_hbm, expr_hbm, table_hbm, w_hbm, b_hbm, out_hbm,
               idx_v, rows_v, out_v, expr_v, w_v, b_v, sem):
    wid = lax.axis_index("s") * NC + lax.axis_index("c")
    row0 = pl.multiple_of(wid * ROWS_PER_W, ROWS_PER_W)

    pltpu.sync_copy(expr_hbm.at[pl.ds(row0, ROWS_PER_W)], expr_v)
    pltpu.sync_copy(w_hbm, w_v)
    pltpu.sync_copy(b_hbm, b_v)

    inv_n = jnp.float32(1.0 / NB_TOKENS)

    def blk_body(blk, carry):
        pair0 = pl.multiple_of((row0 + blk * BLK_ROWS) // PAIR,
                               PAIRS_PER_BLK)
        pltpu.sync_copy(idx_hbm.at[pl.ds(pair0, PAIRS_PER_BLK)], idx_v)
        e_vecs = [expr_v[pl.ds(blk * BLK_ROWS + v * LANES, LANES)]
                  for v in range(BLK_ROWS // LANES)]

        for j in range(PAIRS_PER_BLK):
            pltpu.async_copy(table_hbm.at[idx_v.at[j]], rows_v, sem).wait()
            for rr in range(PAIR):
                base = rr * NB_TOKENS

                def tok(t, accs):
                    return tuple(
                        accs[k] + rows_v[base + t, pl.ds(k * LANES, LANES)]
                        for k in range(DCH)
                    )

                z = jnp.zeros((LANES,), jnp.float32)
                accs = lax.fori_loop(0, NB_TOKENS, tok, (z,) * DCH)
                r_blk = j * PAIR + rr
                e = e_vecs[r_blk // LANES][r_blk % LANES]
                for k in range(DCH):
                    sl = pl.ds(k * LANES, LANES)
                    out_v[r_blk, sl] = accs[k] * inv_n + e * w_v[sl] + b_v[sl]

        out0 = pl.multiple_of(row0 + blk * BLK_ROWS, BLK_ROWS)
        pltpu.sync_copy(out_v, out_hbm.at[pl.ds(out0, BLK_ROWS)])
        return carry

    lax.fori_loop(0, NBLK, blk_body, 0)


@jax.jit
def _run(idx2, expr, table, w, b):
    mesh = plsc.VectorSubcoreMesh(core_axis_name="c", subcore_axis_name="s",
                                  num_cores=NC, num_subcores=NS)
    return pl.kernel(
        _sc_kernel,
        out_type=jax.ShapeDtypeStruct((ROWS, EMBED_DIM), jnp.float32),
        mesh=mesh,
        compiler_params=pltpu.CompilerParams(use_tc_tiling_on_sc=False),
        scratch_types=[
            pltpu.VMEM((PAIRS_PER_BLK, PAIR * NB_TOKENS), jnp.int32),
            pltpu.VMEM((PAIR * NB_TOKENS, EMBED_DIM), jnp.float32),
            pltpu.VMEM((BLK_ROWS, EMBED_DIM), jnp.float32),
            pltpu.VMEM((ROWS_PER_W,), jnp.float32),
            pltpu.VMEM((EMBED_DIM,), jnp.float32),
            pltpu.VMEM((EMBED_DIM,), jnp.float32),
            pltpu.SemaphoreType.DMA,
        ],
    )(idx2, expr, table, w, b)


def kernel(seq_indices, expr_values, emb_table, expr_proj_w, expr_proj_b):
    idx2 = seq_indices.reshape(ROWS // PAIR, PAIR * NB_TOKENS)
    expr = expr_values.reshape(ROWS)
    w = expr_proj_w.reshape(EMBED_DIM)
    return _run(idx2, expr, emb_table, w, expr_proj_b)


# speedup vs baseline: 15.0702x; 1.5513x over previous
import functools

import jax
import jax.numpy as jnp
from jax import lax
from jax.experimental import pallas as pl
from jax.experimental.pallas import tpu as pltpu
from jax.experimental.pallas import tpu_sc as plsc

EMBED_DIM = 64
NB_TOKENS = 50
ROWS = 16384
LANES = 16
NC, NS = 2, 16
NW = NC * NS
ROWS_PER_W = ROWS // NW
PAIR = 2
PAIRS_PER_BLK = 16
BLK_ROWS = PAIR * PAIRS_PER_BLK
NBLK = ROWS_PER_W // BLK_ROWS
DCH = EMBED_DIM // LANES


def _sc_kernel(idx_hbm, expr_hbm, table_hbm, w_hbm, b_hbm, out_hbm,
               idx_v, rows0_v, rows1_v, out_v, expr_v, w_v, b_v,
               sem0, sem1):
    wid = lax.axis_index("s") * NC + lax.axis_index("c")
    row0 = pl.multiple_of(wid * ROWS_PER_W, ROWS_PER_W)

    pltpu.sync_copy(expr_hbm.at[pl.ds(row0, ROWS_PER_W)], expr_v)
    pltpu.sync_copy(w_hbm, w_v)
    pltpu.sync_copy(b_hbm, b_v)

    inv_n = jnp.float32(1.0 / NB_TOKENS)
    bufs = (rows0_v, rows1_v)
    sems = (sem0, sem1)

    def blk_body(blk, carry):
        pair0 = pl.multiple_of((row0 + blk * BLK_ROWS) // PAIR,
                               PAIRS_PER_BLK)
        pltpu.sync_copy(idx_hbm.at[pl.ds(pair0, PAIRS_PER_BLK)], idx_v)
        e_vecs = [expr_v[pl.ds(blk * BLK_ROWS + v * LANES, LANES)]
                  for v in range(BLK_ROWS // LANES)]

        handles = [None] * PAIRS_PER_BLK
        handles[0] = pltpu.async_copy(table_hbm.at[idx_v.at[0]], bufs[0],
                                      sems[0])
        for j in range(PAIRS_PER_BLK):
            if j + 1 < PAIRS_PER_BLK:
                nb = (j + 1) % 2
                handles[j + 1] = pltpu.async_copy(
                    table_hbm.at[idx_v.at[j + 1]], bufs[nb], sems[nb])
            handles[j].wait()
            buf = bufs[j % 2]
            for rr in range(PAIR):
                base = rr * NB_TOKENS

                def tok(t, accs):
                    return tuple(
                        accs[k] + buf[base + t, pl.ds(k * LANES, LANES)]
                        for k in range(DCH)
                    )

                z = jnp.zeros((LANES,), jnp.float32)
                accs = lax.fori_loop(0, NB_TOKENS, tok, (z,) * DCH,
                                     unroll=5)
                r_blk = j * PAIR + rr
                e = e_vecs[r_blk // LANES][r_blk % LANES]
                for k in range(DCH):
                    sl = pl.ds(k * LANES, LANES)
                    out_v[r_blk, sl] = accs[k] * inv_n + e * w_v[sl] + b_v[sl]

        out0 = pl.multiple_of(row0 + blk * BLK_ROWS, BLK_ROWS)
        pltpu.sync_copy(out_v, out_hbm.at[pl.ds(out0, BLK_ROWS)])
        return carry

    lax.fori_loop(0, NBLK, blk_body, 0)


@jax.jit
def _run(idx2, expr, table, w, b):
    mesh = plsc.VectorSubcoreMesh(core_axis_name="c", subcore_axis_name="s",
                                  num_cores=NC, num_subcores=NS)
    return pl.kernel(
        _sc_kernel,
        out_type=jax.ShapeDtypeStruct((ROWS, EMBED_DIM), jnp.float32),
        mesh=mesh,
        compiler_params=pltpu.CompilerParams(use_tc_tiling_on_sc=False),
        scratch_types=[
            pltpu.VMEM((PAIRS_PER_BLK, PAIR * NB_TOKENS), jnp.int32),
            pltpu.VMEM((PAIR * NB_TOKENS, EMBED_DIM), jnp.float32),
            pltpu.VMEM((PAIR * NB_TOKENS, EMBED_DIM), jnp.float32),
            pltpu.VMEM((BLK_ROWS, EMBED_DIM), jnp.float32),
            pltpu.VMEM((ROWS_PER_W,), jnp.float32),
            pltpu.VMEM((EMBED_DIM,), jnp.float32),
            pltpu.VMEM((EMBED_DIM,), jnp.float32),
            pltpu.SemaphoreType.DMA,
            pltpu.SemaphoreType.DMA,
        ],
    )(idx2, expr, table, w, b)


def kernel(seq_indices, expr_values, emb_table, expr_proj_w, expr_proj_b):
    idx2 = seq_indices.reshape(ROWS // PAIR, PAIR * NB_TOKENS)
    expr = expr_values.reshape(ROWS)
    w = expr_proj_w.reshape(EMBED_DIM)
    return _run(idx2, expr, emb_table, w, expr_proj_b)


# speedup vs baseline: 18.4106x; 1.2217x over previous
import functools

import jax
import jax.numpy as jnp
from jax import lax
from jax.experimental import pallas as pl
from jax.experimental.pallas import tpu as pltpu
from jax.experimental.pallas import tpu_sc as plsc

EMBED_DIM = 64
NB_TOKENS = 50
ROWS = 16384
LANES = 16
NC, NS = 2, 16
NW = NC * NS
ROWS_PER_W = ROWS // NW
PAIR = 2
PAIRS_PER_BLK = 16
BLK_ROWS = PAIR * PAIRS_PER_BLK
NBLK = ROWS_PER_W // BLK_ROWS
DCH = EMBED_DIM // LANES


NBUF = 4


def _sc_kernel(idx_hbm, expr_hbm, table_hbm, w_hbm, b_hbm, out_hbm,
               idx_v, rows0_v, rows1_v, rows2_v, rows3_v,
               out_v, expr_v, w_v, b_v, sem0, sem1, sem2, sem3):
    wid = lax.axis_index("s") * NC + lax.axis_index("c")
    row0 = pl.multiple_of(wid * ROWS_PER_W, ROWS_PER_W)

    pltpu.sync_copy(expr_hbm.at[pl.ds(row0, ROWS_PER_W)], expr_v)
    pltpu.sync_copy(w_hbm, w_v)
    pltpu.sync_copy(b_hbm, b_v)

    inv_n = jnp.float32(1.0 / NB_TOKENS)
    bufs = (rows0_v, rows1_v, rows2_v, rows3_v)
    sems = (sem0, sem1, sem2, sem3)

    def blk_body(blk, carry):
        pair0 = pl.multiple_of((row0 + blk * BLK_ROWS) // PAIR,
                               PAIRS_PER_BLK)
        pltpu.sync_copy(idx_hbm.at[pl.ds(pair0, PAIRS_PER_BLK)], idx_v)
        e_vecs = [expr_v[pl.ds(blk * BLK_ROWS + v * LANES, LANES)]
                  for v in range(BLK_ROWS // LANES)]

        handles = [None] * PAIRS_PER_BLK
        for p in range(NBUF - 1):
            handles[p] = pltpu.async_copy(table_hbm.at[idx_v.at[p]],
                                          bufs[p], sems[p])
        for j in range(PAIRS_PER_BLK):
            nj = j + NBUF - 1
            if nj < PAIRS_PER_BLK:
                handles[nj] = pltpu.async_copy(
                    table_hbm.at[idx_v.at[nj]], bufs[nj % NBUF],
                    sems[nj % NBUF])
            handles[j].wait()
            buf = bufs[j % NBUF]
            for rr in range(PAIR):
                base = rr * NB_TOKENS

                def tok(t, accs):
                    return tuple(
                        accs[k] + buf[base + t, pl.ds(k * LANES, LANES)]
                        for k in range(DCH)
                    )

                z = jnp.zeros((LANES,), jnp.float32)
                accs = lax.fori_loop(0, NB_TOKENS, tok, (z,) * DCH,
                                     unroll=5)
                r_blk = j * PAIR + rr
                e = e_vecs[r_blk // LANES][r_blk % LANES]
                for k in range(DCH):
                    sl = pl.ds(k * LANES, LANES)
                    out_v[r_blk, sl] = accs[k] * inv_n + e * w_v[sl] + b_v[sl]

        out0 = pl.multiple_of(row0 + blk * BLK_ROWS, BLK_ROWS)
        pltpu.sync_copy(out_v, out_hbm.at[pl.ds(out0, BLK_ROWS)])
        return carry

    lax.fori_loop(0, NBLK, blk_body, 0)


@jax.jit
def _run(idx2, expr, table, w, b):
    mesh = plsc.VectorSubcoreMesh(core_axis_name="c", subcore_axis_name="s",
                                  num_cores=NC, num_subcores=NS)
    return pl.kernel(
        _sc_kernel,
        out_type=jax.ShapeDtypeStruct((ROWS, EMBED_DIM), jnp.float32),
        mesh=mesh,
        compiler_params=pltpu.CompilerParams(use_tc_tiling_on_sc=False),
        scratch_types=[
            pltpu.VMEM((PAIRS_PER_BLK, PAIR * NB_TOKENS), jnp.int32),
            pltpu.VMEM((PAIR * NB_TOKENS, EMBED_DIM), jnp.float32),
            pltpu.VMEM((PAIR * NB_TOKENS, EMBED_DIM), jnp.float32),
            pltpu.VMEM((PAIR * NB_TOKENS, EMBED_DIM), jnp.float32),
            pltpu.VMEM((PAIR * NB_TOKENS, EMBED_DIM), jnp.float32),
            pltpu.VMEM((BLK_ROWS, EMBED_DIM), jnp.float32),
            pltpu.VMEM((ROWS_PER_W,), jnp.float32),
            pltpu.VMEM((EMBED_DIM,), jnp.float32),
            pltpu.VMEM((EMBED_DIM,), jnp.float32),
            pltpu.SemaphoreType.DMA,
            pltpu.SemaphoreType.DMA,
            pltpu.SemaphoreType.DMA,
            pltpu.SemaphoreType.DMA,
        ],
    )(idx2, expr, table, w, b)


def kernel(seq_indices, expr_values, emb_table, expr_proj_w, expr_proj_b):
    idx2 = seq_indices.reshape(ROWS // PAIR, PAIR * NB_TOKENS)
    expr = expr_values.reshape(ROWS)
    w = expr_proj_w.reshape(EMBED_DIM)
    return _run(idx2, expr, emb_table, w, expr_proj_b)
